# Initial kernel scaffold; baseline (speedup 1.0000x reference)
import jax
import jax.numpy as jnp
from jax.experimental import pallas as pl


def kernel(obj_feats, pairs, W_fuse, b_fuse, W_gcn, W1, b1, W2, b2, W3, b3):
    raise NotImplementedError("write your pallas kernel here")


# speedup vs baseline: 10.4673x; 10.4673x over previous
import jax
import jax.numpy as jnp
from jax.experimental import pallas as pl
from jax.experimental.pallas import tpu as pltpu

_B = 64
_NOBJ = 128
_P = 256
_D = 512
_RCLS = 51
_PAD = 64


def _fused(p0_ref, p1_ref, obj_ref, wft_ref, wfb_ref, bf_ref, wg_ref,
           w1_ref, b1_ref, w2_ref, b2_ref, w3_ref, b3_ref, out_ref):
    p0 = p0_ref[0]
    p1 = p1_ref[0]
    obj = obj_ref[0]
    f32 = jnp.float32
    riota = jax.lax.broadcasted_iota(jnp.int32, (_NOBJ, _P), 0)
    g0t = (riota == jnp.broadcast_to(p0, (_NOBJ, _P))).astype(f32)
    g1t = (riota == jnp.broadcast_to(p1, (_NOBJ, _P))).astype(f32)
    m1t = jnp.where(jnp.broadcast_to(p0 != p1, (_NOBJ, _P)), g1t, 0.0)

    y0 = jnp.dot(obj, wft_ref[...], preferred_element_type=f32)
    y1 = jnp.dot(obj, wfb_ref[...], preferred_element_type=f32)
    z = jnp.dot(obj, wg_ref[...], preferred_element_type=f32)

    dn = (((0,), (0,)), ((), ()))
    init = (jax.lax.dot_general(g0t, y0, dn, preferred_element_type=f32)
            + jax.lax.dot_general(g1t, y1, dn, preferred_element_type=f32)
            + bf_ref[...])
    gcn = jnp.maximum(
        jax.lax.dot_general(g0t + m1t, z, dn, preferred_element_type=f32), 0.0)
    rel = gcn + init

    h = jnp.maximum(jnp.dot(rel, w1_ref[...], preferred_element_type=f32)
                    + b1_ref[...], 0.0)
    h = jnp.maximum(jnp.dot(h, w2_ref[...], preferred_element_type=f32)
                    + b2_ref[...], 0.0)
    dist = jnp.dot(h, w3_ref[...], preferred_element_type=f32) + b3_ref[...]
    m = jnp.max(dist, axis=-1, keepdims=True)
    e = jnp.exp(dist - m)
    out_ref[0] = e / jnp.sum(e, axis=-1, keepdims=True)


def kernel(obj_feats, pairs, W_fuse, b_fuse, W_gcn, W1, b1, W2, b2, W3, b3):
    p = pairs.astype(jnp.int32)
    p0 = p[..., 0].reshape(_B, 1, _P)
    p1 = p[..., 1].reshape(_B, 1, _P)
    wft = W_fuse[:_D]
    wfb = W_fuse[_D:]
    w3p = jnp.zeros((W3.shape[0], _PAD), jnp.float32).at[:, :_RCLS].set(W3)
    b3p = jnp.full((1, _PAD), -1e30, jnp.float32).at[0, :_RCLS].set(b3)

    full = lambda shape: pl.BlockSpec(shape, lambda i: (0,) * len(shape))
    out = pl.pallas_call(
        _fused,
        grid=(_B,),
        in_specs=[
            pl.BlockSpec((1, 1, _P), lambda i: (i, 0, 0)),
            pl.BlockSpec((1, 1, _P), lambda i: (i, 0, 0)),
            pl.BlockSpec((1, _NOBJ, _D), lambda i: (i, 0, 0)),
            full((_D, _D)),
            full((_D, _D)),
            full((1, _D)),
            full((_D, _D)),
            full((_D, 256)),
            full((1, 256)),
            full((256, 128)),
            full((1, 128)),
            full((128, _PAD)),
            full((1, _PAD)),
        ],
        out_specs=pl.BlockSpec((1, _P, _PAD), lambda i: (i, 0, 0)),
        out_shape=jax.ShapeDtypeStruct((_B, _P, _PAD), jnp.float32),
    )(p0, p1, obj_feats, wft, wfb, b_fuse.reshape(1, _D), W_gcn,
      W1, b1.reshape(1, 256), W2, b2.reshape(1, 128), w3p, b3p)
    return out[..., :_RCLS]
